# baseline (speedup 1.0000x reference)
import jax
import jax.numpy as jnp
from jax import lax
from jax.experimental import pallas as pl
from jax.experimental.pallas import tpu as pltpu

_L, _W, _F, _H, _E, _B, _S = 16, 256, 512, 8, 12, 4, 192
_HW = _W // _H
_T = _B * _S


def _dot(a, b):
    return lax.dot_general(a, b, (((1,), (0,)), ((), ())),
                           preferred_element_type=jnp.float32)


def _attn_logits_body(x_ref, Wq_ref, bq_ref, Wk_ref, bk_ref, Wv_ref, bv_ref,
                      logits_ref, v_ref):
    x = x_ref[...]
    Q = _dot(x, Wq_ref[...]) + bq_ref[...]
    K = _dot(x, Wk_ref[...]) + bk_ref[...]
    V = _dot(x, Wv_ref[...]) + bv_ref[...]
    v_ref[...] = V
    row = lax.broadcasted_iota(jnp.int32, (_S, _S), 0)
    col = lax.broadcasted_iota(jnp.int32, (_S, _S), 1)
    mask = jnp.where(col > row, jnp.float32(-jnp.inf), jnp.float32(0.0))
    for b in range(_B):
        for h in range(_H):
            q = Q[b * _S:(b + 1) * _S, h * _HW:(h + 1) * _HW]
            k = K[b * _S:(b + 1) * _S, h * _HW:(h + 1) * _HW]
            a = lax.dot_general(q, k, (((1,), (1,)), ((), ())),
                                preferred_element_type=jnp.float32)
            r = (b * _H + h) * _S
            logits_ref[r:r + _S, :] = a + mask


def _attn_out_body(p_ref, v_ref, x_ref, res_ref):
    x = x_ref[...]
    V = v_ref[...]
    for b in range(_B):
        heads = []
        for h in range(_H):
            r = (b * _H + h) * _S
            ph = p_ref[r:r + _S, :]
            vh = V[b * _S:(b + 1) * _S, h * _HW:(h + 1) * _HW]
            heads.append(_dot(ph, vh))
        res_ref[b * _S:(b + 1) * _S, :] = (
            jnp.concatenate(heads, axis=-1) + x[b * _S:(b + 1) * _S, :])


def _moe_body(norm_ref, gW_ref, gb_ref, W1_ref, b1_ref, WG_ref, bG_ref,
              W2_ref, b2_ref, out_ref, sel_s, acc_s):
    e = pl.program_id(0)
    nrm = norm_ref[...]

    @pl.when(e == 0)
    def _():
        g = _dot(nrm, gW_ref[...]) + gb_ref[...]
        lane = lax.broadcasted_iota(jnp.int32, (_T, _E), 1)
        rank = jnp.zeros((_T, _E), jnp.float32)
        for j in range(_E):
            gj = g[:, j:j + 1]
            better = (gj > g) | ((gj == g) & (j < lane))
            rank = rank + better.astype(jnp.float32)
        sel_s[...] = (rank < 2.0).astype(jnp.float32)
        acc_s[...] = jnp.zeros((_T, _W), jnp.float32)

    h1 = _dot(nrm, W1_ref[0]) + b1_ref[0]
    hg = _dot(nrm, WG_ref[0]) + bG_ref[0]
    hh = h1 * (1.0 / (1.0 + jnp.exp(-h1))) * hg
    o = _dot(hh, W2_ref[0]) + b2_ref[0]
    lane = lax.broadcasted_iota(jnp.int32, (_T, _E), 1)
    msk = jnp.sum(sel_s[...] * (lane == e).astype(jnp.float32),
                  axis=1, keepdims=True)
    acc_s[...] += o * msk

    @pl.when(e == _E - 1)
    def _():
        out_ref[...] = acc_s[...] + nrm


def _attn_logits(x, Wq, bq, Wk, bk, Wv, bv):
    return pl.pallas_call(
        _attn_logits_body,
        out_shape=(jax.ShapeDtypeStruct((_B * _H * _S, _S), jnp.float32),
                   jax.ShapeDtypeStruct((_T, _W), jnp.float32)),
    )(x, Wq, bq, Wk, bk, Wv, bv)


def _attn_out(p, V, x):
    return pl.pallas_call(
        _attn_out_body,
        out_shape=jax.ShapeDtypeStruct((_T, _W), jnp.float32),
    )(p, V, x)


def _moe(norm, gW, gb, W1, b1, WG, bG, W2, b2):
    full = lambda r: pl.BlockSpec(r, lambda e: (0,) * len(r))
    return pl.pallas_call(
        _moe_body,
        grid=(_E,),
        in_specs=[
            full((_T, _W)),
            full((_W, _E)),
            full((1, _E)),
            pl.BlockSpec((1, _W, _F), lambda e: (e, 0, 0)),
            pl.BlockSpec((1, 1, _F), lambda e: (e, 0, 0)),
            pl.BlockSpec((1, _W, _F), lambda e: (e, 0, 0)),
            pl.BlockSpec((1, 1, _F), lambda e: (e, 0, 0)),
            pl.BlockSpec((1, _F, _W), lambda e: (e, 0, 0)),
            pl.BlockSpec((1, 1, _W), lambda e: (e, 0, 0)),
        ],
        out_specs=full((_T, _W)),
        out_shape=jax.ShapeDtypeStruct((_T, _W), jnp.float32),
        scratch_shapes=[
            pltpu.VMEM((_T, _E), jnp.float32),
            pltpu.VMEM((_T, _W), jnp.float32),
        ],
        compiler_params=pltpu.CompilerParams(
            dimension_semantics=("arbitrary",)),
    )(norm, gW, gb, W1, b1, WG, bG, W2, b2)


def _layernorm(x, s, b):
    mu = jnp.mean(x, axis=-1, keepdims=True)
    var = jnp.mean((x - mu) ** 2, axis=-1, keepdims=True)
    return (x - mu) / jnp.sqrt(var + 1e-5) * s + b


def _experts_body(norm_ref, W1_ref, b1_ref, WG_ref, bG_ref, W2_ref, b2_ref,
                  out_ref):
    nrm = norm_ref[...]
    h1 = _dot(nrm, W1_ref[0]) + b1_ref[0]
    hg = _dot(nrm, WG_ref[0]) + bG_ref[0]
    hh = h1 * (1.0 / (1.0 + jnp.exp(-h1))) * hg
    out_ref[...] = _dot(hh, W2_ref[0]) + b2_ref[0]


def _experts(norm, W1, b1, WG, bG, W2, b2):
    return pl.pallas_call(
        _experts_body,
        grid=(_E,),
        in_specs=[
            pl.BlockSpec((_T, _W), lambda e: (0, 0)),
            pl.BlockSpec((1, _W, _F), lambda e: (e, 0, 0)),
            pl.BlockSpec((1, 1, _F), lambda e: (e, 0, 0)),
            pl.BlockSpec((1, _W, _F), lambda e: (e, 0, 0)),
            pl.BlockSpec((1, 1, _F), lambda e: (e, 0, 0)),
            pl.BlockSpec((1, _F, _W), lambda e: (e, 0, 0)),
            pl.BlockSpec((1, 1, _W), lambda e: (e, 0, 0)),
        ],
        out_specs=pl.BlockSpec((_T, _W), lambda e: (0, e)),
        out_shape=jax.ShapeDtypeStruct((_T, _E * _W), jnp.float32),
        compiler_params=pltpu.CompilerParams(
            dimension_semantics=("arbitrary",)),
    )(norm, W1, b1, WG, bG, W2, b2)


_NA = 2 * _T
_BLK = 128
_NB = _NA // _BLK + _E


def _grouped_body(eob_ref, act_ref, xs_ref, W1_ref, b1_ref, WG_ref, bG_ref,
                  W2_ref, b2_ref, ys_ref):
    b = pl.program_id(0)

    @pl.when(act_ref[b] == 1)
    def _():
        xb = xs_ref[...]
        h1 = _dot(xb, W1_ref[0]) + b1_ref[0]
        hg = _dot(xb, WG_ref[0]) + bG_ref[0]
        hh = h1 * (1.0 / (1.0 + jnp.exp(-h1))) * hg
        ys_ref[...] = _dot(hh, W2_ref[0]) + b2_ref[0]


def _grouped(xs, eob, act, W1, b1, WG, bG, W2, b2):
    grid_spec = pltpu.PrefetchScalarGridSpec(
        num_scalar_prefetch=2,
        grid=(_NB,),
        in_specs=[
            pl.BlockSpec((_BLK, _W), lambda b, e_r, a_r: (b, 0)),
            pl.BlockSpec((1, _W, _F), lambda b, e_r, a_r: (e_r[b], 0, 0)),
            pl.BlockSpec((1, 1, _F), lambda b, e_r, a_r: (e_r[b], 0, 0)),
            pl.BlockSpec((1, _W, _F), lambda b, e_r, a_r: (e_r[b], 0, 0)),
            pl.BlockSpec((1, 1, _F), lambda b, e_r, a_r: (e_r[b], 0, 0)),
            pl.BlockSpec((1, _F, _W), lambda b, e_r, a_r: (e_r[b], 0, 0)),
            pl.BlockSpec((1, 1, _W), lambda b, e_r, a_r: (e_r[b], 0, 0)),
        ],
        out_specs=pl.BlockSpec((_BLK, _W), lambda b, e_r, a_r: (b, 0)),
    )
    return pl.pallas_call(
        _grouped_body,
        grid_spec=grid_spec,
        out_shape=jax.ShapeDtypeStruct((_NB * _BLK, _W), jnp.float32),
        compiler_params=pltpu.CompilerParams(
            dimension_semantics=("arbitrary",)),
    )(eob, act, xs, W1, b1, WG, bG, W2, b2)


def _routing_meta(top_idx):
    e_a = top_idx.reshape(_NA)
    order = jnp.argsort(e_a, stable=True)
    inv = jnp.argsort(order)
    sorted_tok = order // 2
    e_sorted = e_a[order]
    eids = jnp.arange(_E, dtype=jnp.int32)
    counts = jnp.sum(e_a[None, :] == eids[:, None], axis=1)
    gstart = jnp.cumsum(counts) - counts
    blocks_e = (counts + _BLK - 1) // _BLK
    bstart = jnp.cumsum(blocks_e) - blocks_e
    nblocks = jnp.sum(blocks_e)
    j = jnp.arange(_NA)
    ppos = _BLK * bstart[e_sorted] + (j - gstart[e_sorted])
    src = jnp.zeros((_NB * _BLK,), jnp.int32).at[ppos].set(
        sorted_tok.astype(jnp.int32))
    dest = ppos[inv]
    blk = jnp.arange(_NB)
    eob = jnp.clip(jnp.searchsorted(bstart, blk, side='right') - 1,
                   0, _E - 1).astype(jnp.int32)
    act = (blk < nblocks).astype(jnp.int32)
    return src, dest, eob, act


def kernel(X, emb, Wq, bq, Wk, bk, Wv, bv, ln1_s, ln1_b, ln2_s, ln2_b,
           gate_W, gate_b, ff1_W, ff1_b, ffG_W, ffG_b, ff2_W, ff2_b):
    x = jnp.take(emb, X[0], axis=0)
    b, s = x.shape[0], x.shape[1]
    ff1_b2 = ff1_b.reshape(_L, _E, 1, _F)
    ffG_b2 = ffG_b.reshape(_L, _E, 1, _F)
    ff2_b2 = ff2_b.reshape(_L, _E, 1, _W)
    mask = jnp.triu(jnp.full((s, s), -jnp.inf, dtype=jnp.float32), 1)
    for l in range(_L):
        Qo = (x @ Wq[l] + bq[l]).reshape(b, s, _H, _HW)
        Ko = (x @ Wk[l] + bk[l]).reshape(b, s, _H, _HW)
        Vo = (x @ Wv[l] + bv[l]).reshape(b, s, _H, _HW)
        att = jnp.einsum('bshw,bShw->bhsS', Qo, Ko) + mask
        att = jax.nn.softmax(att, axis=-1)
        qkv = jnp.einsum('bhsS,bShw->bshw', att, Vo).reshape(b, s, _W)
        norm = _layernorm(qkv + x, ln1_s[l], ln1_b[l])
        gate = jax.nn.softmax(norm @ gate_W[l] + gate_b[l], axis=-1)
        _, top_idx = jax.lax.top_k(gate, 2)
        norm2 = norm.reshape(_T, _W)
        src, dest, eob, act = _routing_meta(top_idx)
        xs = jnp.take(norm2, src, axis=0)
        ys = _grouped(xs, eob, act, ff1_W[l], ff1_b2[l],
                      ffG_W[l], ffG_b2[l], ff2_W[l], ff2_b2[l])
        sel = jnp.take(ys, dest, axis=0).reshape(b, s, 2, _W)
        moe = sel.sum(axis=2)
        x = _layernorm(moe + norm, ln2_s[l], ln2_b[l])
    return x


# speedup vs baseline: 1.6015x; 1.6015x over previous
import jax
import jax.numpy as jnp
from jax import lax
from jax.experimental import pallas as pl
from jax.experimental.pallas import tpu as pltpu

_L, _W, _F, _H, _E, _B, _S = 16, 256, 512, 8, 12, 4, 192
_HW = _W // _H
_T = _B * _S


def _dot(a, b):
    return lax.dot_general(a, b, (((1,), (0,)), ((), ())),
                           preferred_element_type=jnp.float32)


def _attn_logits_body(x_ref, Wq_ref, bq_ref, Wk_ref, bk_ref, Wv_ref, bv_ref,
                      logits_ref, v_ref):
    x = x_ref[...]
    Q = _dot(x, Wq_ref[...]) + bq_ref[...]
    K = _dot(x, Wk_ref[...]) + bk_ref[...]
    V = _dot(x, Wv_ref[...]) + bv_ref[...]
    v_ref[...] = V
    row = lax.broadcasted_iota(jnp.int32, (_S, _S), 0)
    col = lax.broadcasted_iota(jnp.int32, (_S, _S), 1)
    mask = jnp.where(col > row, jnp.float32(-jnp.inf), jnp.float32(0.0))
    for b in range(_B):
        for h in range(_H):
            q = Q[b * _S:(b + 1) * _S, h * _HW:(h + 1) * _HW]
            k = K[b * _S:(b + 1) * _S, h * _HW:(h + 1) * _HW]
            a = lax.dot_general(q, k, (((1,), (1,)), ((), ())),
                                preferred_element_type=jnp.float32)
            r = (b * _H + h) * _S
            logits_ref[r:r + _S, :] = a + mask


def _attn_out_body(p_ref, v_ref, x_ref, res_ref):
    x = x_ref[...]
    V = v_ref[...]
    for b in range(_B):
        heads = []
        for h in range(_H):
            r = (b * _H + h) * _S
            ph = p_ref[r:r + _S, :]
            vh = V[b * _S:(b + 1) * _S, h * _HW:(h + 1) * _HW]
            heads.append(_dot(ph, vh))
        res_ref[b * _S:(b + 1) * _S, :] = (
            jnp.concatenate(heads, axis=-1) + x[b * _S:(b + 1) * _S, :])


def _moe_body(norm_ref, gW_ref, gb_ref, W1_ref, b1_ref, WG_ref, bG_ref,
              W2_ref, b2_ref, out_ref, sel_s, acc_s):
    e = pl.program_id(0)
    nrm = norm_ref[...]

    @pl.when(e == 0)
    def _():
        g = _dot(nrm, gW_ref[...]) + gb_ref[...]
        lane = lax.broadcasted_iota(jnp.int32, (_T, _E), 1)
        rank = jnp.zeros((_T, _E), jnp.float32)
        for j in range(_E):
            gj = g[:, j:j + 1]
            better = (gj > g) | ((gj == g) & (j < lane))
            rank = rank + better.astype(jnp.float32)
        sel_s[...] = (rank < 2.0).astype(jnp.float32)
        acc_s[...] = jnp.zeros((_T, _W), jnp.float32)

    h1 = _dot(nrm, W1_ref[0]) + b1_ref[0]
    hg = _dot(nrm, WG_ref[0]) + bG_ref[0]
    hh = h1 * (1.0 / (1.0 + jnp.exp(-h1))) * hg
    o = _dot(hh, W2_ref[0]) + b2_ref[0]
    lane = lax.broadcasted_iota(jnp.int32, (_T, _E), 1)
    msk = jnp.sum(sel_s[...] * (lane == e).astype(jnp.float32),
                  axis=1, keepdims=True)
    acc_s[...] += o * msk

    @pl.when(e == _E - 1)
    def _():
        out_ref[...] = acc_s[...] + nrm


def _attn_logits(x, Wq, bq, Wk, bk, Wv, bv):
    return pl.pallas_call(
        _attn_logits_body,
        out_shape=(jax.ShapeDtypeStruct((_B * _H * _S, _S), jnp.float32),
                   jax.ShapeDtypeStruct((_T, _W), jnp.float32)),
    )(x, Wq, bq, Wk, bk, Wv, bv)


def _attn_out(p, V, x):
    return pl.pallas_call(
        _attn_out_body,
        out_shape=jax.ShapeDtypeStruct((_T, _W), jnp.float32),
    )(p, V, x)


def _moe(norm, gW, gb, W1, b1, WG, bG, W2, b2):
    full = lambda r: pl.BlockSpec(r, lambda e: (0,) * len(r))
    return pl.pallas_call(
        _moe_body,
        grid=(_E,),
        in_specs=[
            full((_T, _W)),
            full((_W, _E)),
            full((1, _E)),
            pl.BlockSpec((1, _W, _F), lambda e: (e, 0, 0)),
            pl.BlockSpec((1, 1, _F), lambda e: (e, 0, 0)),
            pl.BlockSpec((1, _W, _F), lambda e: (e, 0, 0)),
            pl.BlockSpec((1, 1, _F), lambda e: (e, 0, 0)),
            pl.BlockSpec((1, _F, _W), lambda e: (e, 0, 0)),
            pl.BlockSpec((1, 1, _W), lambda e: (e, 0, 0)),
        ],
        out_specs=full((_T, _W)),
        out_shape=jax.ShapeDtypeStruct((_T, _W), jnp.float32),
        scratch_shapes=[
            pltpu.VMEM((_T, _E), jnp.float32),
            pltpu.VMEM((_T, _W), jnp.float32),
        ],
        compiler_params=pltpu.CompilerParams(
            dimension_semantics=("arbitrary",)),
    )(norm, gW, gb, W1, b1, WG, bG, W2, b2)


def _layernorm(x, s, b):
    mu = jnp.mean(x, axis=-1, keepdims=True)
    var = jnp.mean((x - mu) ** 2, axis=-1, keepdims=True)
    return (x - mu) / jnp.sqrt(var + 1e-5) * s + b


def _experts_body(norm_ref, W1_ref, b1_ref, WG_ref, bG_ref, W2_ref, b2_ref,
                  out_ref):
    e = pl.program_id(0)
    nrm = norm_ref[...]
    h1 = _dot(nrm, W1_ref[0]) + b1_ref[0]
    hg = _dot(nrm, WG_ref[0]) + bG_ref[0]
    hh = h1 * (1.0 / (1.0 + jnp.exp(-h1))) * hg
    o = _dot(hh, W2_ref[0]) + b2_ref[0]
    out_ref[:, pl.ds(e, 1), :] = o[:, None, :]


def _experts(norm, W1, b1, WG, bG, W2, b2):
    return pl.pallas_call(
        _experts_body,
        grid=(_E,),
        in_specs=[
            pl.BlockSpec((_T, _W), lambda e: (0, 0)),
            pl.BlockSpec((1, _W, _F), lambda e: (e, 0, 0)),
            pl.BlockSpec((1, 1, _F), lambda e: (e, 0, 0)),
            pl.BlockSpec((1, _W, _F), lambda e: (e, 0, 0)),
            pl.BlockSpec((1, 1, _F), lambda e: (e, 0, 0)),
            pl.BlockSpec((1, _F, _W), lambda e: (e, 0, 0)),
            pl.BlockSpec((1, 1, _W), lambda e: (e, 0, 0)),
        ],
        out_specs=pl.BlockSpec((_T, _E, _W), lambda e: (0, 0, 0)),
        out_shape=jax.ShapeDtypeStruct((_T, _E, _W), jnp.float32),
        compiler_params=pltpu.CompilerParams(
            dimension_semantics=("arbitrary",)),
    )(norm, W1, b1, WG, bG, W2, b2)


_NA = 2 * _T
_BLK = 128
_NB = _NA // _BLK + _E


def _grouped_body(eob_ref, act_ref, xs_ref, W1_ref, b1_ref, WG_ref, bG_ref,
                  W2_ref, b2_ref, ys_ref):
    b = pl.program_id(0)

    @pl.when(act_ref[b] == 1)
    def _():
        xb = xs_ref[...]
        h1 = _dot(xb, W1_ref[0]) + b1_ref[0]
        hg = _dot(xb, WG_ref[0]) + bG_ref[0]
        hh = h1 * (1.0 / (1.0 + jnp.exp(-h1))) * hg
        ys_ref[...] = _dot(hh, W2_ref[0]) + b2_ref[0]


def _grouped(xs, eob, act, W1, b1, WG, bG, W2, b2):
    grid_spec = pltpu.PrefetchScalarGridSpec(
        num_scalar_prefetch=2,
        grid=(_NB,),
        in_specs=[
            pl.BlockSpec((_BLK, _W), lambda b, e_r, a_r: (b, 0)),
            pl.BlockSpec((1, _W, _F), lambda b, e_r, a_r: (e_r[b], 0, 0)),
            pl.BlockSpec((1, 1, _F), lambda b, e_r, a_r: (e_r[b], 0, 0)),
            pl.BlockSpec((1, _W, _F), lambda b, e_r, a_r: (e_r[b], 0, 0)),
            pl.BlockSpec((1, 1, _F), lambda b, e_r, a_r: (e_r[b], 0, 0)),
            pl.BlockSpec((1, _F, _W), lambda b, e_r, a_r: (e_r[b], 0, 0)),
            pl.BlockSpec((1, 1, _W), lambda b, e_r, a_r: (e_r[b], 0, 0)),
        ],
        out_specs=pl.BlockSpec((_BLK, _W), lambda b, e_r, a_r: (b, 0)),
    )
    return pl.pallas_call(
        _grouped_body,
        grid_spec=grid_spec,
        out_shape=jax.ShapeDtypeStruct((_NB * _BLK, _W), jnp.float32),
        compiler_params=pltpu.CompilerParams(
            dimension_semantics=("arbitrary",)),
    )(eob, act, xs, W1, b1, WG, bG, W2, b2)


def _routing_meta(top_idx):
    e_a = top_idx.reshape(_NA)
    order = jnp.argsort(e_a, stable=True)
    inv = jnp.argsort(order)
    sorted_tok = order // 2
    e_sorted = e_a[order]
    eids = jnp.arange(_E, dtype=jnp.int32)
    counts = jnp.sum(e_a[None, :] == eids[:, None], axis=1)
    gstart = jnp.cumsum(counts) - counts
    blocks_e = (counts + _BLK - 1) // _BLK
    bstart = jnp.cumsum(blocks_e) - blocks_e
    nblocks = jnp.sum(blocks_e)
    j = jnp.arange(_NA)
    ppos = _BLK * bstart[e_sorted] + (j - gstart[e_sorted])
    src = jnp.zeros((_NB * _BLK,), jnp.int32).at[ppos].set(
        sorted_tok.astype(jnp.int32))
    dest = ppos[inv]
    blk = jnp.arange(_NB)
    eob = jnp.clip(jnp.searchsorted(bstart, blk, side='right') - 1,
                   0, _E - 1).astype(jnp.int32)
    act = (blk < nblocks).astype(jnp.int32)
    return src, dest, eob, act


def kernel(X, emb, Wq, bq, Wk, bk, Wv, bv, ln1_s, ln1_b, ln2_s, ln2_b,
           gate_W, gate_b, ff1_W, ff1_b, ffG_W, ffG_b, ff2_W, ff2_b):
    x = jnp.take(emb, X[0], axis=0)
    b, s = x.shape[0], x.shape[1]
    ff1_b2 = ff1_b.reshape(_L, _E, 1, _F)
    ffG_b2 = ffG_b.reshape(_L, _E, 1, _F)
    ff2_b2 = ff2_b.reshape(_L, _E, 1, _W)
    mask = jnp.triu(jnp.full((s, s), -jnp.inf, dtype=jnp.float32), 1)
    for l in range(_L):
        Qo = (x @ Wq[l] + bq[l]).reshape(b, s, _H, _HW)
        Ko = (x @ Wk[l] + bk[l]).reshape(b, s, _H, _HW)
        Vo = (x @ Wv[l] + bv[l]).reshape(b, s, _H, _HW)
        att = jnp.einsum('bshw,bShw->bhsS', Qo, Ko) + mask
        att = jax.nn.softmax(att, axis=-1)
        qkv = jnp.einsum('bhsS,bShw->bshw', att, Vo).reshape(b, s, _W)
        norm = _layernorm(qkv + x, ln1_s[l], ln1_b[l])
        gate = jax.nn.softmax(norm @ gate_W[l] + gate_b[l], axis=-1)
        _, top_idx = jax.lax.top_k(gate, 2)
        exp_out = _experts(norm.reshape(_T, _W), ff1_W[l], ff1_b2[l],
                           ffG_W[l], ffG_b2[l], ff2_W[l],
                           ff2_b2[l]).reshape(b, s, _E, _W)
        sel = jnp.take_along_axis(exp_out, top_idx[..., None], axis=2)
        moe = sel.sum(axis=2)
        x = _layernorm(moe + norm, ln2_s[l], ln2_b[l])
    return x


# speedup vs baseline: 1.6017x; 1.0001x over previous
import jax
import jax.numpy as jnp
from jax import lax
from jax.experimental import pallas as pl
from jax.experimental.pallas import tpu as pltpu

_L, _W, _F, _H, _E, _B, _S = 16, 256, 512, 8, 12, 4, 192
_HW = _W // _H
_T = _B * _S


def _dot(a, b):
    return lax.dot_general(a, b, (((1,), (0,)), ((), ())),
                           preferred_element_type=jnp.float32)


def _experts_body(norm_ref, W1_ref, b1_ref, WG_ref, bG_ref, W2_ref, b2_ref,
                  out_ref):
    e = pl.program_id(0)
    nrm = norm_ref[...]
    h1 = _dot(nrm, W1_ref[0]) + b1_ref[0]
    hg = _dot(nrm, WG_ref[0]) + bG_ref[0]
    hh = h1 * (1.0 / (1.0 + jnp.exp(-h1))) * hg
    o = _dot(hh, W2_ref[0]) + b2_ref[0]
    out_ref[:, pl.ds(e, 1), :] = o[:, None, :]


def _experts(norm, W1, b1, WG, bG, W2, b2):
    return pl.pallas_call(
        _experts_body,
        grid=(_E,),
        in_specs=[
            pl.BlockSpec((_T, _W), lambda e: (0, 0)),
            pl.BlockSpec((1, _W, _F), lambda e: (e, 0, 0)),
            pl.BlockSpec((1, 1, _F), lambda e: (e, 0, 0)),
            pl.BlockSpec((1, _W, _F), lambda e: (e, 0, 0)),
            pl.BlockSpec((1, 1, _F), lambda e: (e, 0, 0)),
            pl.BlockSpec((1, _F, _W), lambda e: (e, 0, 0)),
            pl.BlockSpec((1, 1, _W), lambda e: (e, 0, 0)),
        ],
        out_specs=pl.BlockSpec((_T, _E, _W), lambda e: (0, 0, 0)),
        out_shape=jax.ShapeDtypeStruct((_T, _E, _W), jnp.float32),
        compiler_params=pltpu.CompilerParams(
            dimension_semantics=("arbitrary",)),
    )(norm, W1, b1, WG, bG, W2, b2)


def _layernorm(x, s, b):
    mu = jnp.mean(x, axis=-1, keepdims=True)
    var = jnp.mean((x - mu) ** 2, axis=-1, keepdims=True)
    return (x - mu) / jnp.sqrt(var + 1e-5) * s + b


def kernel(X, emb, Wq, bq, Wk, bk, Wv, bv, ln1_s, ln1_b, ln2_s, ln2_b,
           gate_W, gate_b, ff1_W, ff1_b, ffG_W, ffG_b, ff2_W, ff2_b):
    x = jnp.take(emb, X[0], axis=0)
    b, s = x.shape[0], x.shape[1]
    ff1_b2 = ff1_b.reshape(_L, _E, 1, _F)
    ffG_b2 = ffG_b.reshape(_L, _E, 1, _F)
    ff2_b2 = ff2_b.reshape(_L, _E, 1, _W)
    mask = jnp.triu(jnp.full((s, s), -jnp.inf, dtype=jnp.float32), 1)
    for l in range(_L):
        Qo = (x @ Wq[l] + bq[l]).reshape(b, s, _H, _HW)
        Ko = (x @ Wk[l] + bk[l]).reshape(b, s, _H, _HW)
        Vo = (x @ Wv[l] + bv[l]).reshape(b, s, _H, _HW)
        att = jnp.einsum('bshw,bShw->bhsS', Qo, Ko) + mask
        att = jax.nn.softmax(att, axis=-1)
        qkv = jnp.einsum('bhsS,bShw->bshw', att, Vo).reshape(b, s, _W)
        norm = _layernorm(qkv + x, ln1_s[l], ln1_b[l])
        gate = jax.nn.softmax(norm @ gate_W[l] + gate_b[l], axis=-1)
        _, top_idx = jax.lax.top_k(gate, 2)
        exp_out = _experts(norm.reshape(_T, _W), ff1_W[l], ff1_b2[l],
                           ffG_W[l], ffG_b2[l], ff2_W[l],
                           ff2_b2[l]).reshape(b, s, _E, _W)
        sel = jnp.take_along_axis(exp_out, top_idx[..., None], axis=2)
        moe = sel.sum(axis=2)
        x = _layernorm(moe + norm, ln2_s[l], ln2_b[l])
    return x
